# baseline (speedup 1.0000x reference)
import functools

import jax
import jax.numpy as jnp
from jax import lax
from jax.experimental import pallas as pl
from jax.experimental.pallas import tpu as pltpu
from jax.experimental.pallas import tpu_sc as plsc

_OUT = 7
_NBINS = _OUT * _OUT
_OSTRIDE = 56


def _make_sc_call(N, C, H, W):
    info = plsc.get_sparse_core_info()
    NC, NS = info.num_cores, info.num_subcores
    NW = NC * NS
    assert N % NW == 0
    R = N // NW
    NT = R * _OUT
    assert NT >= 4

    nck = C // 16

    mesh = plsc.VectorSubcoreMesh(core_axis_name="c", subcore_axis_name="s")

    @functools.partial(
        pl.kernel,
        mesh=mesh,
        out_type=jax.ShapeDtypeStruct((N * _OSTRIDE, C), jnp.float32),
        scratch_types=[
            pltpu.VMEM((56,), jnp.int32),
            pltpu.VMEM((5 * W, C), jnp.float32),
            pltpu.VMEM((5 * W, C), jnp.float32),
            pltpu.VMEM((5 * W, C), jnp.float32),
            pltpu.VMEM((3 * 8, C), jnp.float32),
            pltpu.SemaphoreType.DMA,
            pltpu.SemaphoreType.DMA,
            pltpu.SemaphoreType.DMA,
            pltpu.SemaphoreType.DMA,
        ],
    )
    def body(
        feats_hbm,
        rois_hbm,
        out_hbm,
        rois_v,
        buf0,
        buf1,
        buf2,
        obuf,
        sem0,
        sem1,
        sem2,
        semo,
    ):
        wid = lax.axis_index("s") * NC + lax.axis_index("c")
        base = wid * R
        pltpu.sync_copy(rois_hbm.at[pl.ds(base * 5, R * 5)], rois_v.at[pl.ds(0, R * 5)])

        def task_params(t):
            r = t // _OUT
            h = t - r * _OUT
            v = rois_v[pl.ds(r * 5, 16)]
            b = v[0]
            x1 = v[1]
            y1 = v[2]
            x2 = v[3]
            y2 = v[4]
            roi_w = x2 - x1 + 1
            roi_h = y2 - y1 + 1
            rs = y1 + (h * roi_h) // _OUT
            re = y1 + ((h + 1) * roi_h) // _OUT
            return r, h, b, x1, roi_w, rs, re

        def by_height(bh, mk):
            lax.cond(
                bh == 1,
                mk(1),
                lambda: lax.cond(
                    bh == 2,
                    mk(2),
                    lambda: lax.cond(
                        bh == 3, mk(3), lambda: lax.cond(bh == 4, mk(4), mk(5))
                    ),
                ),
            )

        def issue(t, buf, sem):
            @pl.when(t < NT)
            def _():
                r, h, b, x1, roi_w, rs, re = task_params(t)
                start = ((b * H) + rs) * W

                def mk(n):
                    def run():
                        pltpu.async_copy(
                            feats_hbm.at[pl.ds(start, n * W)],
                            buf.at[pl.ds(0, n * W)],
                            sem,
                        )

                    return run

                @pl.when(re > rs)
                def _():
                    by_height(re - rs, mk)

        def consume(t, buf, sem):
            r, h, b, x1, roi_w, rs, re = task_params(t)
            bh = re - rs
            oslot = lax.rem(t, 3) * 8

            @pl.when(t >= 3)
            def _():
                pltpu.make_async_copy(
                    obuf.at[pl.ds(0, 8)],
                    out_hbm.at[pl.ds(0, 8)],
                    semo,
                ).wait()

            def mkd(n):
                def run():
                    pltpu.make_async_copy(
                        feats_hbm.at[pl.ds(0, n * W)],
                        buf.at[pl.ds(0, n * W)],
                        sem,
                    ).wait()

                return run

            @pl.when(bh > 0)
            def _():
                by_height(bh, mkd)

            def wbin_body(w, _):
                ws = (w * roi_w) // _OUT
                we = ((w + 1) * roi_w) // _OUT
                bw = we - ws
                cs = x1 + ws
                bin_i = oslot + w

                def emit(n):
                    def run():
                        if n == 0:
                            z = jnp.zeros((16,), jnp.float32)
                            for k in range(nck):
                                obuf[bin_i, pl.ds(k * 16, 16)] = z
                        else:
                            init = tuple(
                                jnp.full((16,), -jnp.inf, jnp.float32)
                                for _ in range(nck)
                            )

                            def row_body(jr, acc):
                                rbase = jr * W + cs
                                out = acc
                                for tt in range(n):
                                    out = tuple(
                                        jnp.maximum(
                                            out[k],
                                            buf[rbase + tt, pl.ds(k * 16, 16)],
                                        )
                                        for k in range(nck)
                                    )
                                return out

                            acc = lax.fori_loop(0, bh, row_body, init)
                            ne = bh > 0
                            for k in range(nck):
                                obuf[bin_i, pl.ds(k * 16, 16)] = jnp.where(
                                    ne, acc[k], 0.0
                                )

                    return run

                lax.cond(
                    bw == 0,
                    emit(0),
                    lambda: lax.cond(
                        bw == 1,
                        emit(1),
                        lambda: lax.cond(
                            bw == 2,
                            emit(2),
                            lambda: lax.cond(
                                bw == 3,
                                emit(3),
                                lambda: lax.cond(bw == 4, emit(4), emit(5)),
                            ),
                        ),
                    ),
                )
                return 0

            lax.fori_loop(0, _OUT, wbin_body, 0)

            pltpu.async_copy(
                obuf.at[pl.ds(oslot, 8)],
                out_hbm.at[pl.ds((base + r) * _OSTRIDE + h * 8, 8)],
                semo,
            )

        def consume_g(t, buf, sem):
            @pl.when(t < NT)
            def _():
                consume(t, buf, sem)

        issue(0, buf0, sem0)
        issue(1, buf1, sem1)

        def k_body(k, _):
            t = 3 * k
            issue(t + 2, buf2, sem2)
            consume_g(t, buf0, sem0)
            issue(t + 3, buf0, sem0)
            consume_g(t + 1, buf1, sem1)
            issue(t + 4, buf1, sem1)
            consume_g(t + 2, buf2, sem2)
            return 0

        lax.fori_loop(0, (NT + 2) // 3, k_body, 0)

        for _ in range(3):
            pltpu.make_async_copy(
                obuf.at[pl.ds(0, 8)],
                out_hbm.at[pl.ds(0, 8)],
                semo,
            ).wait()

    return body


def kernel(features, rois):
    B, C, H, W = features.shape
    N = rois.shape[0]
    feats = jnp.transpose(features, (0, 2, 3, 1)).reshape(B * H * W, C)
    rois_flat = rois.astype(jnp.int32).reshape(N * 5)
    out = _make_sc_call(N, C, H, W)(feats, rois_flat)
    out = out.reshape(N, _OUT, 8, C)[:, :, :_OUT]
    return out.transpose(0, 3, 1, 2).reshape(N, C, _OUT, _OUT)


# speedup vs baseline: 1.1611x; 1.1611x over previous
import functools

import jax
import jax.numpy as jnp
from jax import lax
from jax.experimental import pallas as pl
from jax.experimental.pallas import tpu as pltpu
from jax.experimental.pallas import tpu_sc as plsc

_OUT = 7
_NBINS = _OUT * _OUT
_OSTRIDE = 56


def _make_sc_call(N, C, H, W):
    info = plsc.get_sparse_core_info()
    NC, NS = info.num_cores, info.num_subcores
    NW = NC * NS
    assert N % NW == 0
    R = N // NW
    assert R >= 2 and (R * _OUT) % 2 == 0
    NT = R * _OUT

    nck = C // 16

    mesh = plsc.VectorSubcoreMesh(core_axis_name="c", subcore_axis_name="s")

    @functools.partial(
        pl.kernel,
        mesh=mesh,
        out_type=jax.ShapeDtypeStruct((N * _OSTRIDE, C), jnp.float32),
        scratch_types=[
            pltpu.VMEM((64,), jnp.int32),
            pltpu.VMEM((5 * W, C), jnp.float32),
            pltpu.VMEM((5 * W, C), jnp.float32),
            pltpu.VMEM((2 * _OSTRIDE, C), jnp.float32),
            pltpu.SemaphoreType.DMA,
            pltpu.SemaphoreType.DMA,
            pltpu.SemaphoreType.DMA,
        ],
    )
    def body(feats_hbm, rois_hbm, out_hbm, rois_v, buf0, buf1, obuf, sem0, sem1, semo):
        wid = lax.axis_index("s") * NC + lax.axis_index("c")
        base = wid * R
        pltpu.sync_copy(rois_hbm.at[pl.ds(base * 5, R * 5)], rois_v.at[pl.ds(0, R * 5)])

        def task_params(t):
            r = t // _OUT
            h = t - r * _OUT
            v = rois_v[pl.ds(r * 5, 16)]
            b = v[0]
            x1 = v[1]
            y1 = v[2]
            x2 = v[3]
            y2 = v[4]
            roi_w = x2 - x1 + 1
            roi_h = y2 - y1 + 1
            rs = y1 + (h * roi_h) // _OUT
            re = y1 + ((h + 1) * roi_h) // _OUT
            return r, h, b, x1, roi_w, rs, re

        def by_height(bh, mk):
            lax.cond(
                bh == 1,
                mk(1),
                lambda: lax.cond(
                    bh == 2,
                    mk(2),
                    lambda: lax.cond(
                        bh == 3, mk(3), lambda: lax.cond(bh == 4, mk(4), mk(5))
                    ),
                ),
            )

        def issue(t, buf, sem):
            @pl.when(t < NT)
            def _():
                r, h, b, x1, roi_w, rs, re = task_params(t)
                start = ((b * H) + rs) * W

                def mk(n):
                    def run():
                        pltpu.async_copy(
                            feats_hbm.at[pl.ds(start, n * W)],
                            buf.at[pl.ds(0, n * W)],
                            sem,
                        )

                    return run

                @pl.when(re > rs)
                def _():
                    by_height(re - rs, mk)

        def consume(t, buf, sem):
            r, h, b, x1, roi_w, rs, re = task_params(t)
            bh = re - rs
            oslot = (r % 2) * _OSTRIDE

            @pl.when((h == 0) & (r >= 2))
            def _():
                pltpu.make_async_copy(
                    obuf.at[pl.ds(0, _OSTRIDE)],
                    out_hbm.at[pl.ds(0, _OSTRIDE)],
                    semo,
                ).wait()

            def mkd(n):
                def run():
                    pltpu.make_async_copy(
                        feats_hbm.at[pl.ds(0, n * W)],
                        buf.at[pl.ds(0, n * W)],
                        sem,
                    ).wait()

                return run

            @pl.when(bh > 0)
            def _():
                by_height(bh, mkd)

            def wbin_body(w, _):
                ws = (w * roi_w) // _OUT
                we = ((w + 1) * roi_w) // _OUT
                bw = we - ws
                cs = x1 + ws
                bin_i = oslot + h * _OUT + w

                def emit(n):
                    def run():
                        if n == 0:
                            z = jnp.zeros((16,), jnp.float32)
                            for k in range(nck):
                                obuf[bin_i, pl.ds(k * 16, 16)] = z
                        else:
                            init = tuple(
                                jnp.full((16,), -jnp.inf, jnp.float32)
                                for _ in range(nck)
                            )

                            def row_body(jr, acc):
                                rbase = jr * W + cs
                                out = acc
                                for tt in range(n):
                                    out = tuple(
                                        jnp.maximum(
                                            out[k],
                                            buf[rbase + tt, pl.ds(k * 16, 16)],
                                        )
                                        for k in range(nck)
                                    )
                                return out

                            acc = lax.fori_loop(0, bh, row_body, init)
                            ne = bh > 0
                            for k in range(nck):
                                obuf[bin_i, pl.ds(k * 16, 16)] = jnp.where(
                                    ne, acc[k], 0.0
                                )

                    return run

                lax.cond(
                    bw == 0,
                    emit(0),
                    lambda: lax.cond(
                        bw == 1,
                        emit(1),
                        lambda: lax.cond(
                            bw == 2,
                            emit(2),
                            lambda: lax.cond(
                                bw == 3,
                                emit(3),
                                lambda: lax.cond(bw == 4, emit(4), emit(5)),
                            ),
                        ),
                    ),
                )
                return 0

            lax.fori_loop(0, _OUT, wbin_body, 0)

            @pl.when(h == _OUT - 1)
            def _():
                pltpu.async_copy(
                    obuf.at[pl.ds(oslot, _OSTRIDE)],
                    out_hbm.at[pl.ds((base + r) * _OSTRIDE, _OSTRIDE)],
                    semo,
                )

        issue(0, buf0, sem0)

        def k_body(k, _):
            t = 2 * k
            issue(t + 1, buf1, sem1)
            consume(t, buf0, sem0)
            issue(t + 2, buf0, sem0)
            consume(t + 1, buf1, sem1)
            return 0

        lax.fori_loop(0, NT // 2, k_body, 0)

        for _ in range(2):
            pltpu.make_async_copy(
                obuf.at[pl.ds(0, _OSTRIDE)],
                out_hbm.at[pl.ds(0, _OSTRIDE)],
                semo,
            ).wait()

    return body


def kernel(features, rois):
    B, C, H, W = features.shape
    N = rois.shape[0]
    feats = jnp.transpose(features, (0, 2, 3, 1)).reshape(B * H * W, C)
    rois_flat = rois.astype(jnp.int32).reshape(N * 5)
    out = _make_sc_call(N, C, H, W)(feats, rois_flat)
    out = out.reshape(N, _OSTRIDE, C)[:, :_NBINS]
    return out.transpose(0, 2, 1).reshape(N, C, _OUT, _OUT)
